# Initial kernel scaffold; baseline (speedup 1.0000x reference)
import jax
import jax.numpy as jnp
from jax.experimental import pallas as pl


def kernel(category, target_category, table, W1, b1, W2, b2):
    raise NotImplementedError("write your pallas kernel here")


# speedup vs baseline: 2.9626x; 2.9626x over previous
import functools

import jax
import jax.numpy as jnp
from jax import lax
from jax.experimental import pallas as pl
from jax.experimental.pallas import tpu as pltpu, tpu_sc as plsc

NUM_CAT = 13
D_CAT = 16
N_SUB = 5

_NC = 2
_NS = 16
_NW = _NC * _NS
_L = 16


def _pair_table_body(t_ref, w1_ref, b1_ref, w2_ref, b2_ref, p_ref):
    t = t_ref[:]
    norm = jnp.sqrt(jnp.sum(t * t, axis=1, keepdims=True))
    n = t / jnp.maximum(norm, 1e-12)
    w1 = w1_ref[:]
    b1 = b1_ref[:]
    w2 = w2_ref[:]
    b2 = b2_ref[:]
    for i in range(16):
        e = n + n[i : i + 1, :]
        h = jnp.maximum(
            jnp.dot(e, w1, preferred_element_type=jnp.float32) + b1, 0.0
        )
        g = jnp.dot(h, w2, preferred_element_type=jnp.float32) + b2
        m = jnp.max(g, axis=1, keepdims=True)
        ex = jnp.exp(g - m)
        sm = ex / jnp.sum(ex, axis=1, keepdims=True)
        p_ref[i * 16 : (i + 1) * 16, :] = sm


def _make_sc_lookup(batch):
    bpw = batch // _NW
    groups = bpw // _L
    mesh = plsc.VectorSubcoreMesh(core_axis_name="c", subcore_axis_name="s")

    @functools.partial(
        pl.kernel,
        mesh=mesh,
        out_type=jax.ShapeDtypeStruct((batch * N_SUB,), jnp.float32),
        scratch_types=[
            pltpu.VMEM((4096,), jnp.float32),
            pltpu.VMEM((bpw,), jnp.int32),
            pltpu.VMEM((bpw,), jnp.int32),
            pltpu.VMEM((bpw * N_SUB,), jnp.float32),
        ],
        compiler_params=pltpu.CompilerParams(needs_layout_passes=False),
    )
    def sc_lookup(p_hbm, cat_hbm, tcat_hbm, out_hbm, p_v, cat_v, tcat_v, out_v):
        wid = lax.axis_index("s") * _NC + lax.axis_index("c")
        base = wid * bpw
        pltpu.sync_copy(p_hbm, p_v)
        pltpu.sync_copy(cat_hbm.at[pl.ds(base, bpw)], cat_v)
        pltpu.sync_copy(tcat_hbm.at[pl.ds(base, bpw)], tcat_v)
        lane = lax.iota(jnp.int32, _L)
        zero = jnp.zeros((_L,), jnp.int32)
        topc = jnp.full((_L,), NUM_CAT - 1, jnp.int32)
        for g in range(groups):
            c = cat_v[pl.ds(g * _L, _L)]
            t = tcat_v[pl.ds(g * _L, _L)]
            c = jnp.minimum(jnp.maximum(c, zero), topc)
            t = jnp.minimum(jnp.maximum(t, zero), topc)
            row = c * 16 + t
            for s in range(N_SUB):
                vals = plsc.load_gather(p_v, [row * 16 + s])
                plsc.store_scatter(
                    out_v, [lane * N_SUB + (g * _L * N_SUB + s)], vals
                )
        pltpu.sync_copy(out_v, out_hbm.at[pl.ds(base * N_SUB, bpw * N_SUB)])

    return sc_lookup


def kernel(category, target_category, table, W1, b1, W2, b2):
    f32 = jnp.float32
    batch = category.shape[0]
    table16 = jnp.zeros((16, 16), f32).at[:NUM_CAT, :].set(table.astype(f32))
    b1r = b1.astype(f32).reshape(1, D_CAT)
    w2p = jnp.zeros((16, 16), f32).at[:, :N_SUB].set(W2.astype(f32))
    b2p = jnp.full((1, 16), -1e30, f32).at[0, :N_SUB].set(b2.astype(f32))

    pair_table = pl.pallas_call(
        _pair_table_body,
        out_shape=jax.ShapeDtypeStruct((256, 16), f32),
    )(table16, W1.astype(f32), b1r, w2p, b2p)

    cat = category.astype(jnp.int32)
    tcat = target_category.astype(jnp.int32)
    out_flat = _make_sc_lookup(batch)(pair_table.reshape(4096), cat, tcat)
    return out_flat.reshape(batch, N_SUB)


# speedup vs baseline: 3.2636x; 1.1016x over previous
import functools

import jax
import jax.numpy as jnp
from jax import lax
from jax.experimental import pallas as pl
from jax.experimental.pallas import tpu as pltpu, tpu_sc as plsc

NUM_CAT = 13
D_CAT = 16
N_SUB = 5

_NC = 2
_NS = 16
_NW = _NC * _NS
_L = 16


def _pair_table_body(t_ref, w1_ref, b1_ref, w2_ref, b2_ref, p_ref):
    t13 = t_ref[:]
    t = jnp.concatenate([t13, jnp.zeros((3, 16), jnp.float32)], axis=0)
    norm = jnp.sqrt(jnp.sum(t * t, axis=1, keepdims=True))
    n = t / jnp.maximum(norm, 1e-12)
    w1 = w1_ref[:]
    b1 = jnp.reshape(b1_ref[:], (1, 16))
    w2 = jnp.concatenate(
        [w2_ref[:], jnp.zeros((16, 11), jnp.float32)], axis=1
    )
    b2 = jnp.reshape(
        jnp.concatenate([b2_ref[:], jnp.full((11,), -1e30, jnp.float32)]),
        (1, 16),
    )
    for i in range(16):
        e = n + n[i : i + 1, :]
        h = jnp.maximum(
            jnp.dot(e, w1, preferred_element_type=jnp.float32) + b1, 0.0
        )
        g = jnp.dot(h, w2, preferred_element_type=jnp.float32) + b2
        m = jnp.max(g, axis=1, keepdims=True)
        ex = jnp.exp(g - m)
        sm = ex / jnp.sum(ex, axis=1, keepdims=True)
        p_ref[pl.ds(i * 16, 16), :] = sm


def _make_sc_lookup(batch):
    bpw = batch // _NW
    groups = bpw // _L
    mesh = plsc.VectorSubcoreMesh(core_axis_name="c", subcore_axis_name="s")

    @functools.partial(
        pl.kernel,
        mesh=mesh,
        out_type=jax.ShapeDtypeStruct((batch * N_SUB,), jnp.float32),
        scratch_types=[
            pltpu.VMEM((4096,), jnp.float32),
            pltpu.VMEM((bpw,), jnp.int32),
            pltpu.VMEM((bpw,), jnp.int32),
            pltpu.VMEM((bpw * N_SUB,), jnp.float32),
        ],
        compiler_params=pltpu.CompilerParams(needs_layout_passes=False),
    )
    def sc_lookup(p_hbm, cat_hbm, tcat_hbm, out_hbm, p_v, cat_v, tcat_v, out_v):
        wid = lax.axis_index("s") * _NC + lax.axis_index("c")
        base = wid * bpw
        pltpu.sync_copy(p_hbm, p_v)
        pltpu.sync_copy(cat_hbm.at[pl.ds(base, bpw)], cat_v)
        pltpu.sync_copy(tcat_hbm.at[pl.ds(base, bpw)], tcat_v)
        lane = lax.iota(jnp.int32, _L)
        zero = jnp.zeros((_L,), jnp.int32)
        topc = jnp.full((_L,), NUM_CAT - 1, jnp.int32)
        for g in range(groups):
            c = cat_v[pl.ds(g * _L, _L)]
            t = tcat_v[pl.ds(g * _L, _L)]
            c = jnp.minimum(jnp.maximum(c, zero), topc)
            t = jnp.minimum(jnp.maximum(t, zero), topc)
            row = c * 16 + t
            for s in range(N_SUB):
                vals = plsc.load_gather(p_v, [row * 16 + s])
                plsc.store_scatter(
                    out_v, [lane * N_SUB + (g * _L * N_SUB + s)], vals
                )
        pltpu.sync_copy(out_v, out_hbm.at[pl.ds(base * N_SUB, bpw * N_SUB)])

    return sc_lookup


def kernel(category, target_category, table, W1, b1, W2, b2):
    f32 = jnp.float32
    batch = category.shape[0]

    pair_table = pl.pallas_call(
        _pair_table_body,
        out_shape=jax.ShapeDtypeStruct((256, 16), f32),
    )(table.astype(f32), W1.astype(f32), b1.astype(f32), W2.astype(f32),
      b2.astype(f32))

    cat = category.astype(jnp.int32)
    tcat = target_category.astype(jnp.int32)
    out_flat = _make_sc_lookup(batch)(pair_table.reshape(4096), cat, tcat)
    return out_flat.reshape(batch, N_SUB)
